# Initial kernel scaffold; baseline (speedup 1.0000x reference)
import jax
import jax.numpy as jnp
from jax.experimental import pallas as pl


def kernel(x, W):
    raise NotImplementedError("write your pallas kernel here")


# speedup vs baseline: 3.0821x; 3.0821x over previous
import functools

import jax
import jax.numpy as jnp
import numpy as np
from jax import lax
from jax.experimental import pallas as pl
from jax.experimental.pallas import tpu as pltpu
from jax.experimental.pallas import tpu_sc as plsc

_FIELD_DIM = 3846
_F = 26
_V = _F * _FIELD_DIM
_D = 16
_NPAIRS = (_F * (_F - 1)) // 2
_PAIR_PAD = 336

_pi = np.zeros(_PAIR_PAD, np.int32)
_pj = np.zeros(_PAIR_PAD, np.int32)
_p = 0
for _i in range(_F - 1):
    for _j in range(_i + 1, _F):
        _pi[_p], _pj[_p] = _i, _j
        _p += 1


def _body(x_hbm, wf_hbm, ii_hbm, jj_hbm, out_hbm,
          x_v, ii_v, jj_v, idxa_v, idxb_v, bufa, bufb, bufc, sema, semb):
    info = plsc.get_sparse_core_info()
    nc = info.num_cores
    c = x_v.shape[0]
    wid = lax.axis_index("s") * nc + lax.axis_index("c")
    base = wid * c
    pltpu.sync_copy(x_hbm.at[pl.ds(base, c), :], x_v)
    pltpu.sync_copy(ii_hbm, ii_v)
    pltpu.sync_copy(jj_hbm, jj_v)
    iota = lax.iota(jnp.int32, 16)

    def pair_body(p, carry):
        pvec = jnp.full((16,), p, jnp.int32)
        i_vec = plsc.load_gather(ii_v, [pvec])
        j_vec = plsc.load_gather(jj_v, [pvec])
        for v in range(c // 16):
            lanes = iota + (16 * v)
            xa = plsc.load_gather(x_v, [lanes, j_vec])
            xb = plsc.load_gather(x_v, [lanes, i_vec])
            idxa_v[pl.ds(16 * v, 16)] = i_vec * _V + j_vec * _FIELD_DIM + xa
            idxb_v[pl.ds(16 * v, 16)] = j_vec * _V + i_vec * _FIELD_DIM + xb
        cpa = pltpu.async_copy(wf_hbm.at[idxa_v], bufa, sema)
        cpb = pltpu.async_copy(wf_hbm.at[idxb_v], bufb, semb)
        cpa.wait()
        cpb.wait()
        for r in range(c):
            bufc[r, :] = bufa[r, :] * bufb[r, :]
        pltpu.sync_copy(bufc, out_hbm.at[pl.ds(base, c), p])
        return carry

    lax.fori_loop(0, _NPAIRS, pair_body, 0)


def kernel(x, W):
    b, f = x.shape
    assert f == _F
    wf = W.reshape(_F * _V, _D)
    x32 = x.astype(jnp.int32)
    info = plsc.get_sparse_core_info()
    nw = info.num_cores * info.num_subcores
    c = b // nw
    mesh = plsc.VectorSubcoreMesh(core_axis_name="c", subcore_axis_name="s")
    kfn = functools.partial(
        pl.kernel,
        out_type=jax.ShapeDtypeStruct((b, _NPAIRS, _D), jnp.float32),
        mesh=mesh,
        compiler_params=pltpu.CompilerParams(needs_layout_passes=False,
                                             use_tc_tiling_on_sc=False),
        scratch_types=[
            pltpu.VMEM((c, _F), jnp.int32),
            pltpu.VMEM((_PAIR_PAD,), jnp.int32),
            pltpu.VMEM((_PAIR_PAD,), jnp.int32),
            pltpu.VMEM((c,), jnp.int32),
            pltpu.VMEM((c,), jnp.int32),
            pltpu.VMEM((c, _D), jnp.float32),
            pltpu.VMEM((c, _D), jnp.float32),
            pltpu.VMEM((c, _D), jnp.float32),
            pltpu.SemaphoreType.DMA,
            pltpu.SemaphoreType.DMA,
        ],
    )(_body)
    return kfn(x32, wf, jnp.asarray(_pi), jnp.asarray(_pj))


# speedup vs baseline: 3.2019x; 1.0389x over previous
import functools

import jax
import jax.numpy as jnp
import numpy as np
from jax import lax
from jax.experimental import pallas as pl
from jax.experimental.pallas import tpu as pltpu
from jax.experimental.pallas import tpu_sc as plsc

_FIELD_DIM = 3846
_F = 26
_V = _F * _FIELD_DIM
_D = 16
_NPAIRS = (_F * (_F - 1)) // 2
_PAIR_PAD = 336
_NBUF = 5
_NGROUPS = _NPAIRS // _NBUF

_pi = np.zeros(_PAIR_PAD, np.int32)
_pj = np.zeros(_PAIR_PAD, np.int32)
_p = 0
for _i in range(_F - 1):
    for _j in range(_i + 1, _F):
        _pi[_p], _pj[_p] = _i, _j
        _p += 1


def _body(x_hbm, wf_hbm, ii_hbm, jj_hbm, out_hbm, x_v, ii_v, jj_v, *slots):
    idxa = slots[0:_NBUF]
    idxb = slots[_NBUF:2 * _NBUF]
    bufa = slots[2 * _NBUF:3 * _NBUF]
    bufb = slots[3 * _NBUF:4 * _NBUF]
    bufc = slots[4 * _NBUF:5 * _NBUF]
    sema = slots[5 * _NBUF:6 * _NBUF]
    semb = slots[6 * _NBUF:7 * _NBUF]
    semo = slots[7 * _NBUF:8 * _NBUF]

    info = plsc.get_sparse_core_info()
    nc = info.num_cores
    c = x_v.shape[0]
    wid = lax.axis_index("s") * nc + lax.axis_index("c")
    base = wid * c
    pltpu.sync_copy(x_hbm.at[pl.ds(base, c), :], x_v)
    pltpu.sync_copy(ii_hbm, ii_v)
    pltpu.sync_copy(jj_hbm, jj_v)
    iota = lax.iota(jnp.int32, 16)

    def build_and_fire(i_vec, j_vec, s):
        for v in range(c // 16):
            lanes = iota + (16 * v)
            xa = plsc.load_gather(x_v, [lanes, j_vec])
            xb = plsc.load_gather(x_v, [lanes, i_vec])
            idxa[s][pl.ds(16 * v, 16)] = i_vec * _V + j_vec * _FIELD_DIM + xa
            idxb[s][pl.ds(16 * v, 16)] = j_vec * _V + i_vec * _FIELD_DIM + xb
        pltpu.async_copy(wf_hbm.at[idxa[s]], bufa[s], sema[s])
        pltpu.async_copy(wf_hbm.at[idxb[s]], bufb[s], semb[s])

    def issue_static(p, s):
        i_vec = jnp.full((16,), int(_pi[p]), jnp.int32)
        j_vec = jnp.full((16,), int(_pj[p]), jnp.int32)
        build_and_fire(i_vec, j_vec, s)

    def issue_dyn(p, s):
        pvec = jnp.full((16,), p, jnp.int32)
        i_vec = plsc.load_gather(ii_v, [pvec])
        j_vec = plsc.load_gather(jj_v, [pvec])
        build_and_fire(i_vec, j_vec, s)

    def wait_gathers(s):
        pltpu.make_async_copy(wf_hbm.at[idxa[s]], bufa[s], sema[s]).wait()
        pltpu.make_async_copy(wf_hbm.at[idxb[s]], bufb[s], semb[s]).wait()

    def multiply_and_flush(p, s):
        for r in range(c):
            bufc[s][r, :] = bufa[s][r, :] * bufb[s][r, :]
        pltpu.async_copy(bufc[s], out_hbm.at[pl.ds(base, c), p], semo[s])

    def wait_out(p, s):
        pltpu.make_async_copy(
            bufc[s], out_hbm.at[pl.ds(base, c), p], semo[s]).wait()

    for s in range(_NBUF):
        issue_static(s, s)

    for s in range(_NBUF):
        wait_gathers(s)
        multiply_and_flush(s, s)
        issue_static(s + _NBUF, s)

    def group_body(g, carry):
        for s in range(_NBUF):
            p = g * _NBUF + s
            wait_gathers(s)
            wait_out(p, s)
            multiply_and_flush(p, s)
            issue_dyn(p + _NBUF, s)
        return carry

    lax.fori_loop(1, _NGROUPS, group_body, 0)
    for s in range(_NBUF):
        wait_gathers(s)
        wait_out(s, s)


def kernel(x, W):
    b, f = x.shape
    assert f == _F
    wf = W.reshape(_F * _V, _D)
    x32 = x.astype(jnp.int32)
    info = plsc.get_sparse_core_info()
    nw = info.num_cores * info.num_subcores
    c = b // nw
    mesh = plsc.VectorSubcoreMesh(core_axis_name="c", subcore_axis_name="s")
    scratch = [
        pltpu.VMEM((c, _F), jnp.int32),
        pltpu.VMEM((_PAIR_PAD,), jnp.int32),
        pltpu.VMEM((_PAIR_PAD,), jnp.int32),
    ]
    scratch += [pltpu.VMEM((c,), jnp.int32) for _ in range(2 * _NBUF)]
    scratch += [pltpu.VMEM((c, _D), jnp.float32) for _ in range(3 * _NBUF)]
    scratch += [pltpu.SemaphoreType.DMA for _ in range(3 * _NBUF)]
    kfn = functools.partial(
        pl.kernel,
        out_type=jax.ShapeDtypeStruct((b, _NPAIRS, _D), jnp.float32),
        mesh=mesh,
        compiler_params=pltpu.CompilerParams(needs_layout_passes=False,
                                             use_tc_tiling_on_sc=False),
        scratch_types=scratch,
    )(_body)
    return kfn(x32, wf, jnp.asarray(_pi), jnp.asarray(_pj))


# speedup vs baseline: 5.8400x; 1.8239x over previous
import functools

import jax
import jax.numpy as jnp
import numpy as np
from jax import lax
from jax.experimental import pallas as pl
from jax.experimental.pallas import tpu as pltpu
from jax.experimental.pallas import tpu_sc as plsc

_FIELD_DIM = 3846
_F = 26
_V = _F * _FIELD_DIM
_VP = 100352
_D = 16
_NPAIRS = (_F * (_F - 1)) // 2
_PAIR_PAD = 336
_NBUF = 5
_NGROUPS = _NPAIRS // _NBUF

_pi = np.zeros(_PAIR_PAD, np.int32)
_pj = np.zeros(_PAIR_PAD, np.int32)
_p = 0
for _i in range(_F - 1):
    for _j in range(_i + 1, _F):
        _pi[_p], _pj[_p] = _i, _j
        _p += 1


def _w_body(ws_ref, out_ref):
    out_ref[...] = ws_ref[0].T


def _reformat_w(W):
    ws = jnp.swapaxes(W, 1, 2)
    return pl.pallas_call(
        _w_body,
        grid=(_F, _VP // 512),
        in_specs=[pl.BlockSpec((1, _D, 512), lambda i, v: (i, 0, v))],
        out_specs=pl.BlockSpec((512, _D), lambda i, v: (i * (_VP // 512) + v, 0)),
        out_shape=jax.ShapeDtypeStruct((_F * _VP, _D), jnp.float32),
    )(ws)


def _sc_body(x_hbm, wf_hbm, ii_hbm, jj_hbm, out_hbm, x_v, ii_v, jj_v, *slots):
    idxa = slots[0:_NBUF]
    idxb = slots[_NBUF:2 * _NBUF]
    bufa = slots[2 * _NBUF:3 * _NBUF]
    bufb = slots[3 * _NBUF:4 * _NBUF]
    tbuf = slots[4 * _NBUF:5 * _NBUF]
    sema = slots[5 * _NBUF:6 * _NBUF]
    semb = slots[6 * _NBUF:7 * _NBUF]
    semo = slots[7 * _NBUF:8 * _NBUF]

    info = plsc.get_sparse_core_info()
    nc = info.num_cores
    c = x_v.shape[0]
    wid = lax.axis_index("s") * nc + lax.axis_index("c")
    base = wid * c
    base8 = wid * 8
    pltpu.sync_copy(x_hbm.at[pl.ds(base, c), :], x_v)
    pltpu.sync_copy(ii_hbm, ii_v)
    pltpu.sync_copy(jj_hbm, jj_v)
    iota = lax.iota(jnp.int32, 16)

    def build_and_fire(i_vec, j_vec, s):
        for v in range(c // 16):
            lanes = iota + (16 * v)
            xa = plsc.load_gather(x_v, [lanes, j_vec])
            xb = plsc.load_gather(x_v, [lanes, i_vec])
            idxa[s][pl.ds(16 * v, 16)] = i_vec * _VP + j_vec * _FIELD_DIM + xa
            idxb[s][pl.ds(16 * v, 16)] = j_vec * _VP + i_vec * _FIELD_DIM + xb
        pltpu.async_copy(wf_hbm.at[idxa[s]], bufa[s], sema[s])
        pltpu.async_copy(wf_hbm.at[idxb[s]], bufb[s], semb[s])

    def issue_static(p, s):
        i_vec = jnp.full((16,), int(_pi[p]), jnp.int32)
        j_vec = jnp.full((16,), int(_pj[p]), jnp.int32)
        build_and_fire(i_vec, j_vec, s)

    def issue_dyn(p, s):
        pvec = jnp.full((16,), p, jnp.int32)
        i_vec = plsc.load_gather(ii_v, [pvec])
        j_vec = plsc.load_gather(jj_v, [pvec])
        build_and_fire(i_vec, j_vec, s)

    def wait_gathers(s):
        pltpu.make_async_copy(wf_hbm.at[idxa[s]], bufa[s], sema[s]).wait()
        pltpu.make_async_copy(wf_hbm.at[idxb[s]], bufb[s], semb[s]).wait()

    def multiply_and_flush(p, s):
        for r in range(c):
            prod = bufa[s][r, :] * bufb[s][r, :]
            plsc.store_scatter(
                tbuf[s], [iota, jnp.full((16,), r, jnp.int32)], prod)
        q0 = p * 512 + base8
        pltpu.async_copy(
            tbuf[s].at[pl.ds(0, 8), :], out_hbm.at[pl.ds(q0, 8), :], semo[s])
        pltpu.async_copy(
            tbuf[s].at[pl.ds(8, 8), :],
            out_hbm.at[pl.ds(q0 + 256, 8), :], semo[s])

    def wait_out(p, s):
        q0 = p * 512 + base8
        pltpu.make_async_copy(
            tbuf[s].at[pl.ds(0, 8), :], out_hbm.at[pl.ds(q0, 8), :],
            semo[s]).wait()
        pltpu.make_async_copy(
            tbuf[s].at[pl.ds(8, 8), :], out_hbm.at[pl.ds(q0 + 256, 8), :],
            semo[s]).wait()

    for s in range(_NBUF):
        issue_static(s, s)

    for s in range(_NBUF):
        wait_gathers(s)
        multiply_and_flush(s, s)
        issue_static(s + _NBUF, s)

    def group_body(g, carry):
        for s in range(_NBUF):
            p = g * _NBUF + s
            wait_gathers(s)
            wait_out(p, s)
            multiply_and_flush(p, s)
            issue_dyn(p + _NBUF, s)
        return carry

    lax.fori_loop(1, _NGROUPS, group_body, 0)
    for s in range(_NBUF):
        wait_gathers(s)
        wait_out(s, s)


def kernel(x, W):
    b, f = x.shape
    assert f == _F
    wf = _reformat_w(W)
    x32 = x.astype(jnp.int32)
    info = plsc.get_sparse_core_info()
    nw = info.num_cores * info.num_subcores
    c = b // nw
    nrows = b * _NPAIRS * _D // 128
    mesh = plsc.VectorSubcoreMesh(core_axis_name="c", subcore_axis_name="s")
    scratch = [
        pltpu.VMEM((c, _F), jnp.int32),
        pltpu.VMEM((_PAIR_PAD,), jnp.int32),
        pltpu.VMEM((_PAIR_PAD,), jnp.int32),
    ]
    scratch += [pltpu.VMEM((c,), jnp.int32) for _ in range(2 * _NBUF)]
    scratch += [pltpu.VMEM((c, _D), jnp.float32) for _ in range(2 * _NBUF)]
    scratch += [pltpu.VMEM((_D, 128), jnp.float32) for _ in range(_NBUF)]
    scratch += [pltpu.SemaphoreType.DMA for _ in range(3 * _NBUF)]
    kfn = functools.partial(
        pl.kernel,
        out_type=jax.ShapeDtypeStruct((nrows, 128), jnp.float32),
        mesh=mesh,
        compiler_params=pltpu.CompilerParams(needs_layout_passes=False,
                                             use_tc_tiling_on_sc=False),
        scratch_types=scratch,
    )(_sc_body)
    out2d = kfn(x32, wf, jnp.asarray(_pi), jnp.asarray(_pj))
    return (out2d.reshape(_NPAIRS, 2, b // 128, 8, 128)
            .transpose(2, 4, 0, 1, 3)
            .reshape(b, _NPAIRS, _D))


# speedup vs baseline: 7.0517x; 1.2075x over previous
import functools

import jax
import jax.numpy as jnp
import numpy as np
from jax import lax
from jax.experimental import pallas as pl
from jax.experimental.pallas import tpu as pltpu
from jax.experimental.pallas import tpu_sc as plsc

_FIELD_DIM = 3846
_F = 26
_V = _F * _FIELD_DIM
_VP = 100352
_D = 16
_NPAIRS = (_F * (_F - 1)) // 2
_PAIR_PAD = 336
_NBUF = 5
_NGROUPS = _NPAIRS // _NBUF

_pi = np.zeros(_PAIR_PAD, np.int32)
_pj = np.zeros(_PAIR_PAD, np.int32)
_p = 0
for _i in range(_F - 1):
    for _j in range(_i + 1, _F):
        _pi[_p], _pj[_p] = _i, _j
        _p += 1


def _w_body(ws_ref, out_ref):
    t3 = ws_ref[0].T.reshape(64, 8, _D)
    out_ref[...] = jnp.concatenate([t3[:, e, :] for e in range(8)], axis=1)


def _reformat_w(W):
    ws = jnp.swapaxes(W, 1, 2)
    out128 = pl.pallas_call(
        _w_body,
        grid=(_F, _VP // 512),
        in_specs=[pl.BlockSpec((1, _D, 512), lambda i, v: (i, 0, v))],
        out_specs=pl.BlockSpec((64, 128), lambda i, v: (i * (_VP // 512) + v, 0)),
        out_shape=jax.ShapeDtypeStruct((_F * _VP * _D // 128, 128), jnp.float32),
    )(ws)
    return out128.reshape(_F * _VP, _D)


def _sc_body(x_hbm, wf_hbm, ii_hbm, jj_hbm, out_hbm, x_v, ii_v, jj_v, *slots):
    idxa = slots[0:_NBUF]
    idxb = slots[_NBUF:2 * _NBUF]
    bufa = slots[2 * _NBUF:3 * _NBUF]
    bufb = slots[3 * _NBUF:4 * _NBUF]
    tbuf = slots[4 * _NBUF:5 * _NBUF]
    sema = slots[5 * _NBUF:6 * _NBUF]
    semb = slots[6 * _NBUF:7 * _NBUF]
    semo = slots[7 * _NBUF:8 * _NBUF]

    info = plsc.get_sparse_core_info()
    nc = info.num_cores
    c = x_v.shape[0]
    wid = lax.axis_index("s") * nc + lax.axis_index("c")
    base = wid * c
    base8 = wid * 8
    pltpu.sync_copy(x_hbm.at[pl.ds(base, c), :], x_v)
    pltpu.sync_copy(ii_hbm, ii_v)
    pltpu.sync_copy(jj_hbm, jj_v)
    iota = lax.iota(jnp.int32, 16)

    def build_and_fire(i_vec, j_vec, s):
        for v in range(c // 16):
            lanes = iota + (16 * v)
            xa = plsc.load_gather(x_v, [lanes, j_vec])
            xb = plsc.load_gather(x_v, [lanes, i_vec])
            idxa[s][pl.ds(16 * v, 16)] = i_vec * _VP + j_vec * _FIELD_DIM + xa
            idxb[s][pl.ds(16 * v, 16)] = j_vec * _VP + i_vec * _FIELD_DIM + xb
        pltpu.async_copy(wf_hbm.at[idxa[s]], bufa[s], sema[s])
        pltpu.async_copy(wf_hbm.at[idxb[s]], bufb[s], semb[s])

    def issue_static(p, s):
        i_vec = jnp.full((16,), int(_pi[p]), jnp.int32)
        j_vec = jnp.full((16,), int(_pj[p]), jnp.int32)
        build_and_fire(i_vec, j_vec, s)

    def issue_dyn(p, s):
        pvec = jnp.full((16,), p, jnp.int32)
        i_vec = plsc.load_gather(ii_v, [pvec])
        j_vec = plsc.load_gather(jj_v, [pvec])
        build_and_fire(i_vec, j_vec, s)

    def wait_gathers(s):
        pltpu.make_async_copy(wf_hbm.at[idxa[s]], bufa[s], sema[s]).wait()
        pltpu.make_async_copy(wf_hbm.at[idxb[s]], bufb[s], semb[s]).wait()

    def multiply_and_flush(p, s):
        for r in range(c):
            prod = bufa[s][r, :] * bufb[s][r, :]
            plsc.store_scatter(
                tbuf[s], [iota, jnp.full((16,), r, jnp.int32)], prod)
        q0 = p * 512 + base8
        pltpu.async_copy(
            tbuf[s].at[pl.ds(0, 8), :], out_hbm.at[pl.ds(q0, 8), :], semo[s])
        pltpu.async_copy(
            tbuf[s].at[pl.ds(8, 8), :],
            out_hbm.at[pl.ds(q0 + 256, 8), :], semo[s])

    def wait_out(p, s):
        q0 = p * 512 + base8
        pltpu.make_async_copy(
            tbuf[s].at[pl.ds(0, 8), :], out_hbm.at[pl.ds(q0, 8), :],
            semo[s]).wait()
        pltpu.make_async_copy(
            tbuf[s].at[pl.ds(8, 8), :], out_hbm.at[pl.ds(q0 + 256, 8), :],
            semo[s]).wait()

    for s in range(_NBUF):
        issue_static(s, s)

    for s in range(_NBUF):
        wait_gathers(s)
        multiply_and_flush(s, s)
        issue_static(s + _NBUF, s)

    def group_body(g, carry):
        for s in range(_NBUF):
            p = g * _NBUF + s
            wait_gathers(s)
            wait_out(p, s)
            multiply_and_flush(p, s)
            issue_dyn(p + _NBUF, s)
        return carry

    lax.fori_loop(1, _NGROUPS, group_body, 0)
    for s in range(_NBUF):
        wait_gathers(s)
        wait_out(s, s)


def kernel(x, W):
    b, f = x.shape
    assert f == _F
    wf = _reformat_w(W)
    x32 = x.astype(jnp.int32)
    info = plsc.get_sparse_core_info()
    nw = info.num_cores * info.num_subcores
    c = b // nw
    nrows = b * _NPAIRS * _D // 128
    mesh = plsc.VectorSubcoreMesh(core_axis_name="c", subcore_axis_name="s")
    scratch = [
        pltpu.VMEM((c, _F), jnp.int32),
        pltpu.VMEM((_PAIR_PAD,), jnp.int32),
        pltpu.VMEM((_PAIR_PAD,), jnp.int32),
    ]
    scratch += [pltpu.VMEM((c,), jnp.int32) for _ in range(2 * _NBUF)]
    scratch += [pltpu.VMEM((c, _D), jnp.float32) for _ in range(2 * _NBUF)]
    scratch += [pltpu.VMEM((_D, 128), jnp.float32) for _ in range(_NBUF)]
    scratch += [pltpu.SemaphoreType.DMA for _ in range(3 * _NBUF)]
    kfn = functools.partial(
        pl.kernel,
        out_type=jax.ShapeDtypeStruct((nrows, 128), jnp.float32),
        mesh=mesh,
        compiler_params=pltpu.CompilerParams(needs_layout_passes=False,
                                             use_tc_tiling_on_sc=False),
        scratch_types=scratch,
    )(_sc_body)
    out2d = kfn(x32, wf, jnp.asarray(_pi), jnp.asarray(_pj))
    return (out2d.reshape(_NPAIRS, 2, b // 128, 8, 128)
            .transpose(2, 4, 0, 1, 3)
            .reshape(b, _NPAIRS, _D))


# speedup vs baseline: 18.0119x; 2.5543x over previous
import functools

import jax
import jax.numpy as jnp
import numpy as np
from jax import lax
from jax.experimental import pallas as pl
from jax.experimental.pallas import tpu as pltpu
from jax.experimental.pallas import tpu_sc as plsc

_FIELD_DIM = 3846
_F = 26
_V = _F * _FIELD_DIM
_VP = 100352
_D = 16
_NPAIRS = (_F * (_F - 1)) // 2
_PAIR_PAD = 336
_NBUF = 5
_NGROUPS = _NPAIRS // _NBUF

_pi = np.zeros(_PAIR_PAD, np.int32)
_pj = np.zeros(_PAIR_PAD, np.int32)
_p = 0
for _i in range(_F - 1):
    for _j in range(_i + 1, _F):
        _pi[_p], _pj[_p] = _i, _j
        _p += 1


def _w_body(ws_ref, out_ref):
    t3 = ws_ref[0].T.reshape(896, 8, _D)
    out_ref[...] = jnp.concatenate([t3[:, e, :] for e in range(8)], axis=1)


def _reformat_w(W):
    ws = jnp.swapaxes(W, 1, 2)
    out128 = pl.pallas_call(
        _w_body,
        grid=(_F, _VP // 7168),
        in_specs=[pl.BlockSpec((1, _D, 7168), lambda i, v: (i, 0, v))],
        out_specs=pl.BlockSpec((896, 128), lambda i, v: (i * (_VP // 7168) + v, 0)),
        out_shape=jax.ShapeDtypeStruct((_F * _VP * _D // 128, 128), jnp.float32),
    )(ws)
    return out128.reshape(_F * _VP, _D)


def _sc_body(x_hbm, wf_hbm, ii_hbm, jj_hbm, out_hbm, x_v, ii_v, jj_v, *slots):
    idxa = slots[0:_NBUF]
    idxb = slots[_NBUF:2 * _NBUF]
    bufa = slots[2 * _NBUF:3 * _NBUF]
    bufb = slots[3 * _NBUF:4 * _NBUF]
    tbuf = slots[4 * _NBUF:5 * _NBUF]
    sema = slots[5 * _NBUF:6 * _NBUF]
    semb = slots[6 * _NBUF:7 * _NBUF]
    semo = slots[7 * _NBUF:8 * _NBUF]

    info = plsc.get_sparse_core_info()
    nc = info.num_cores
    c = x_v.shape[0]
    wid = lax.axis_index("s") * nc + lax.axis_index("c")
    base = wid * c
    base8 = wid * 8
    pltpu.sync_copy(x_hbm.at[pl.ds(base, c), :], x_v)
    pltpu.sync_copy(ii_hbm, ii_v)
    pltpu.sync_copy(jj_hbm, jj_v)
    iota = lax.iota(jnp.int32, 16)

    def build_and_fire(i_vec, j_vec, s):
        for v in range(c // 16):
            lanes = iota + (16 * v)
            xa = plsc.load_gather(x_v, [lanes, j_vec])
            xb = plsc.load_gather(x_v, [lanes, i_vec])
            idxa[s][pl.ds(16 * v, 16)] = i_vec * _VP + j_vec * _FIELD_DIM + xa
            idxb[s][pl.ds(16 * v, 16)] = j_vec * _VP + i_vec * _FIELD_DIM + xb
        pltpu.async_copy(wf_hbm.at[idxa[s]], bufa[s], sema[s])
        pltpu.async_copy(wf_hbm.at[idxb[s]], bufb[s], semb[s])

    def issue_static(p, s):
        i_vec = jnp.full((16,), int(_pi[p]), jnp.int32)
        j_vec = jnp.full((16,), int(_pj[p]), jnp.int32)
        build_and_fire(i_vec, j_vec, s)

    def issue_dyn(p, s):
        pvec = jnp.full((16,), p, jnp.int32)
        i_vec = plsc.load_gather(ii_v, [pvec])
        j_vec = plsc.load_gather(jj_v, [pvec])
        build_and_fire(i_vec, j_vec, s)

    def wait_gathers(s):
        pltpu.make_async_copy(wf_hbm.at[idxa[s]], bufa[s], sema[s]).wait()
        pltpu.make_async_copy(wf_hbm.at[idxb[s]], bufb[s], semb[s]).wait()

    def multiply_and_flush(p, s):
        for r in range(c):
            prod = bufa[s][r, :] * bufb[s][r, :]
            plsc.store_scatter(
                tbuf[s], [iota, jnp.full((16,), r, jnp.int32)], prod)
        q0 = p * 512 + base8
        pltpu.async_copy(
            tbuf[s].at[pl.ds(0, 8), :], out_hbm.at[pl.ds(q0, 8), :], semo[s])
        pltpu.async_copy(
            tbuf[s].at[pl.ds(8, 8), :],
            out_hbm.at[pl.ds(q0 + 256, 8), :], semo[s])

    def wait_out(p, s):
        q0 = p * 512 + base8
        pltpu.make_async_copy(
            tbuf[s].at[pl.ds(0, 8), :], out_hbm.at[pl.ds(q0, 8), :],
            semo[s]).wait()
        pltpu.make_async_copy(
            tbuf[s].at[pl.ds(8, 8), :], out_hbm.at[pl.ds(q0 + 256, 8), :],
            semo[s]).wait()

    for s in range(_NBUF):
        issue_static(s, s)

    for s in range(_NBUF):
        wait_gathers(s)
        multiply_and_flush(s, s)
        issue_static(s + _NBUF, s)

    def group_body(g, carry):
        for s in range(_NBUF):
            p = g * _NBUF + s
            wait_gathers(s)
            wait_out(p, s)
            multiply_and_flush(p, s)
            issue_dyn(p + _NBUF, s)
        return carry

    lax.fori_loop(1, _NGROUPS, group_body, 0)
    for s in range(_NBUF):
        wait_gathers(s)
        wait_out(s, s)


def kernel(x, W):
    b, f = x.shape
    assert f == _F
    wf = _reformat_w(W)
    x32 = x.astype(jnp.int32)
    info = plsc.get_sparse_core_info()
    nw = info.num_cores * info.num_subcores
    c = b // nw
    nrows = b * _NPAIRS * _D // 128
    mesh = plsc.VectorSubcoreMesh(core_axis_name="c", subcore_axis_name="s")
    scratch = [
        pltpu.VMEM((c, _F), jnp.int32),
        pltpu.VMEM((_PAIR_PAD,), jnp.int32),
        pltpu.VMEM((_PAIR_PAD,), jnp.int32),
    ]
    scratch += [pltpu.VMEM((c,), jnp.int32) for _ in range(2 * _NBUF)]
    scratch += [pltpu.VMEM((c, _D), jnp.float32) for _ in range(2 * _NBUF)]
    scratch += [pltpu.VMEM((_D, 128), jnp.float32) for _ in range(_NBUF)]
    scratch += [pltpu.SemaphoreType.DMA for _ in range(3 * _NBUF)]
    kfn = functools.partial(
        pl.kernel,
        out_type=jax.ShapeDtypeStruct((nrows, 128), jnp.float32),
        mesh=mesh,
        compiler_params=pltpu.CompilerParams(needs_layout_passes=False,
                                             use_tc_tiling_on_sc=False),
        scratch_types=scratch,
    )(_sc_body)
    out2d = kfn(x32, wf, jnp.asarray(_pi), jnp.asarray(_pj))
    return (out2d.reshape(_NPAIRS, 2, b // 128, 8, 128)
            .transpose(2, 4, 0, 1, 3)
            .reshape(b, _NPAIRS, _D))


# speedup vs baseline: 22.4216x; 1.2448x over previous
import functools

import jax
import jax.numpy as jnp
import numpy as np
from jax import lax
from jax.experimental import pallas as pl
from jax.experimental.pallas import tpu as pltpu
from jax.experimental.pallas import tpu_sc as plsc

_FIELD_DIM = 3846
_F = 26
_V = _F * _FIELD_DIM
_VP = 100352
_D = 16
_NPAIRS = (_F * (_F - 1)) // 2
_PAIR_PAD = 336
_NBUF = 5
_NGROUPS = _NPAIRS // _NBUF

_pi = np.zeros(_PAIR_PAD, np.int32)
_pj = np.zeros(_PAIR_PAD, np.int32)
_p = 0
for _i in range(_F - 1):
    for _j in range(_i + 1, _F):
        _pi[_p], _pj[_p] = _i, _j
        _p += 1


def _w_body(ws_ref, out_ref):
    t3 = ws_ref[0].T.reshape(896, 8, _D)
    for e in range(8):
        out_ref[:, 16 * e:16 * e + 16] = t3[:, e, :]


def _reformat_w(W):
    ws = jnp.swapaxes(W, 1, 2)
    out128 = pl.pallas_call(
        _w_body,
        grid=(_F, _VP // 7168),
        in_specs=[pl.BlockSpec((1, _D, 7168), lambda i, v: (i, 0, v))],
        out_specs=pl.BlockSpec((896, 128), lambda i, v: (i * (_VP // 7168) + v, 0)),
        out_shape=jax.ShapeDtypeStruct((_F * _VP * _D // 128, 128), jnp.float32),
    )(ws)
    return out128.reshape(_F * _VP, _D)


def _sc_body(x_hbm, wf_hbm, ii_hbm, jj_hbm, out_hbm, x_v, ii_v, jj_v, *slots):
    idxa = slots[0:_NBUF]
    idxb = slots[_NBUF:2 * _NBUF]
    bufa = slots[2 * _NBUF:3 * _NBUF]
    bufb = slots[3 * _NBUF:4 * _NBUF]
    tbuf = slots[4 * _NBUF:5 * _NBUF]
    sema = slots[5 * _NBUF:6 * _NBUF]
    semb = slots[6 * _NBUF:7 * _NBUF]
    semo = slots[7 * _NBUF:8 * _NBUF]

    info = plsc.get_sparse_core_info()
    nc = info.num_cores
    c = x_v.shape[0]
    wid = lax.axis_index("s") * nc + lax.axis_index("c")
    base = wid * c
    base8 = wid * 8
    pltpu.sync_copy(x_hbm.at[pl.ds(base, c), :], x_v)
    pltpu.sync_copy(ii_hbm, ii_v)
    pltpu.sync_copy(jj_hbm, jj_v)
    iota = lax.iota(jnp.int32, 16)

    def build_and_fire(i_vec, j_vec, s):
        for v in range(c // 16):
            lanes = iota + (16 * v)
            xa = plsc.load_gather(x_v, [lanes, j_vec])
            xb = plsc.load_gather(x_v, [lanes, i_vec])
            idxa[s][pl.ds(16 * v, 16)] = i_vec * _VP + j_vec * _FIELD_DIM + xa
            idxb[s][pl.ds(16 * v, 16)] = j_vec * _VP + i_vec * _FIELD_DIM + xb
        pltpu.async_copy(wf_hbm.at[idxa[s]], bufa[s], sema[s])
        pltpu.async_copy(wf_hbm.at[idxb[s]], bufb[s], semb[s])

    def issue_static(p, s):
        i_vec = jnp.full((16,), int(_pi[p]), jnp.int32)
        j_vec = jnp.full((16,), int(_pj[p]), jnp.int32)
        build_and_fire(i_vec, j_vec, s)

    def issue_dyn(p, s):
        pvec = jnp.full((16,), p, jnp.int32)
        i_vec = plsc.load_gather(ii_v, [pvec])
        j_vec = plsc.load_gather(jj_v, [pvec])
        build_and_fire(i_vec, j_vec, s)

    def wait_gathers(s):
        pltpu.make_async_copy(wf_hbm.at[idxa[s]], bufa[s], sema[s]).wait()
        pltpu.make_async_copy(wf_hbm.at[idxb[s]], bufb[s], semb[s]).wait()

    def multiply_and_flush(p, s):
        for r in range(c):
            prod = bufa[s][r, :] * bufb[s][r, :]
            plsc.store_scatter(
                tbuf[s], [iota, jnp.full((16,), r, jnp.int32)], prod)
        q0 = p * 512 + base8
        pltpu.async_copy(
            tbuf[s].at[pl.ds(0, 8), pl.ds(0, 128)],
            out_hbm.at[pl.ds(q0, 8), :], semo[s])
        pltpu.async_copy(
            tbuf[s].at[pl.ds(8, 8), pl.ds(0, 128)],
            out_hbm.at[pl.ds(q0 + 256, 8), :], semo[s])

    def wait_out(p, s):
        q0 = p * 512 + base8
        pltpu.make_async_copy(
            tbuf[s].at[pl.ds(0, 8), pl.ds(0, 128)],
            out_hbm.at[pl.ds(q0, 8), :], semo[s]).wait()
        pltpu.make_async_copy(
            tbuf[s].at[pl.ds(8, 8), pl.ds(0, 128)],
            out_hbm.at[pl.ds(q0 + 256, 8), :], semo[s]).wait()

    for s in range(_NBUF):
        issue_static(s, s)

    for s in range(_NBUF):
        wait_gathers(s)
        multiply_and_flush(s, s)
        issue_static(s + _NBUF, s)

    def group_body(g, carry):
        for s in range(_NBUF):
            p = g * _NBUF + s
            wait_gathers(s)
            wait_out(p, s)
            multiply_and_flush(p, s)
            issue_dyn(p + _NBUF, s)
        return carry

    lax.fori_loop(1, _NGROUPS, group_body, 0)
    for s in range(_NBUF):
        wait_gathers(s)
        wait_out(s, s)


def kernel(x, W):
    b, f = x.shape
    assert f == _F
    wf = _reformat_w(W)
    x32 = x.astype(jnp.int32)
    info = plsc.get_sparse_core_info()
    nw = info.num_cores * info.num_subcores
    c = b // nw
    nrows = b * _NPAIRS * _D // 128
    mesh = plsc.VectorSubcoreMesh(core_axis_name="c", subcore_axis_name="s")
    scratch = [
        pltpu.VMEM((c, _F), jnp.int32),
        pltpu.VMEM((_PAIR_PAD,), jnp.int32),
        pltpu.VMEM((_PAIR_PAD,), jnp.int32),
    ]
    scratch += [pltpu.VMEM((c,), jnp.int32) for _ in range(2 * _NBUF)]
    scratch += [pltpu.VMEM((c, _D), jnp.float32) for _ in range(2 * _NBUF)]
    scratch += [pltpu.VMEM((_D, 129), jnp.float32) for _ in range(_NBUF)]
    scratch += [pltpu.SemaphoreType.DMA for _ in range(3 * _NBUF)]
    kfn = functools.partial(
        pl.kernel,
        out_type=jax.ShapeDtypeStruct((nrows, 128), jnp.float32),
        mesh=mesh,
        compiler_params=pltpu.CompilerParams(needs_layout_passes=False,
                                             use_tc_tiling_on_sc=False),
        scratch_types=scratch,
    )(_sc_body)
    out2d = kfn(x32, wf, jnp.asarray(_pi), jnp.asarray(_pj))
    return (out2d.reshape(_NPAIRS, 2, b // 128, 8, 128)
            .transpose(2, 4, 0, 1, 3)
            .reshape(b, _NPAIRS, _D))


# speedup vs baseline: 22.8691x; 1.0200x over previous
import functools

import jax
import jax.numpy as jnp
import numpy as np
from jax import lax
from jax.experimental import pallas as pl
from jax.experimental.pallas import tpu as pltpu
from jax.experimental.pallas import tpu_sc as plsc

_FIELD_DIM = 3846
_F = 26
_V = _F * _FIELD_DIM
_VP = 100352
_D = 16
_NPAIRS = (_F * (_F - 1)) // 2
_PAIR_PAD = 336
_NBUF = 5
_NGROUPS = _NPAIRS // _NBUF

_pi = np.zeros(_PAIR_PAD, np.int32)
_pj = np.zeros(_PAIR_PAD, np.int32)
_p = 0
for _i in range(_F - 1):
    for _j in range(_i + 1, _F):
        _pi[_p], _pj[_p] = _i, _j
        _p += 1


def _w_body(ws_ref, out_ref):
    t3 = ws_ref[0].T.reshape(1792, 8, _D)
    for e in range(8):
        out_ref[:, 16 * e:16 * e + 16] = t3[:, e, :]


def _reformat_w(W):
    ws = jnp.swapaxes(W, 1, 2)
    out128 = pl.pallas_call(
        _w_body,
        grid=(_F, _VP // 14336),
        in_specs=[pl.BlockSpec((1, _D, 14336), lambda i, v: (i, 0, v))],
        out_specs=pl.BlockSpec((1792, 128), lambda i, v: (i * (_VP // 14336) + v, 0)),
        out_shape=jax.ShapeDtypeStruct((_F * _VP * _D // 128, 128), jnp.float32),
    )(ws)
    return out128.reshape(_F * _VP, _D)


def _sc_body(x_hbm, wf_hbm, ii_hbm, jj_hbm, out_hbm, x_v, ii_v, jj_v, *slots):
    idxa = slots[0:_NBUF]
    idxb = slots[_NBUF:2 * _NBUF]
    bufa = slots[2 * _NBUF:3 * _NBUF]
    bufb = slots[3 * _NBUF:4 * _NBUF]
    tbuf = slots[4 * _NBUF:5 * _NBUF]
    sema = slots[5 * _NBUF:6 * _NBUF]
    semb = slots[6 * _NBUF:7 * _NBUF]
    semo = slots[7 * _NBUF:8 * _NBUF]

    info = plsc.get_sparse_core_info()
    nc = info.num_cores
    c = x_v.shape[0]
    wid = lax.axis_index("s") * nc + lax.axis_index("c")
    base = wid * c
    base8 = wid * 8
    pltpu.sync_copy(x_hbm.at[pl.ds(base, c), :], x_v)
    pltpu.sync_copy(ii_hbm, ii_v)
    pltpu.sync_copy(jj_hbm, jj_v)
    iota = lax.iota(jnp.int32, 16)

    def build_and_fire(i_vec, j_vec, s):
        for v in range(c // 16):
            lanes = iota + (16 * v)
            xa = plsc.load_gather(x_v, [lanes, j_vec])
            xb = plsc.load_gather(x_v, [lanes, i_vec])
            idxa[s][pl.ds(16 * v, 16)] = i_vec * _VP + j_vec * _FIELD_DIM + xa
            idxb[s][pl.ds(16 * v, 16)] = j_vec * _VP + i_vec * _FIELD_DIM + xb
        pltpu.async_copy(wf_hbm.at[idxa[s]], bufa[s], sema[s])
        pltpu.async_copy(wf_hbm.at[idxb[s]], bufb[s], semb[s])

    def issue_static(p, s):
        i_vec = jnp.full((16,), int(_pi[p]), jnp.int32)
        j_vec = jnp.full((16,), int(_pj[p]), jnp.int32)
        build_and_fire(i_vec, j_vec, s)

    def issue_dyn(p, s):
        pvec = jnp.full((16,), p, jnp.int32)
        i_vec = plsc.load_gather(ii_v, [pvec])
        j_vec = plsc.load_gather(jj_v, [pvec])
        build_and_fire(i_vec, j_vec, s)

    def wait_gathers(s):
        pltpu.make_async_copy(wf_hbm.at[idxa[s]], bufa[s], sema[s]).wait()
        pltpu.make_async_copy(wf_hbm.at[idxb[s]], bufb[s], semb[s]).wait()

    def multiply_and_flush(p, s):
        for r in range(c):
            prod = bufa[s][r, :] * bufb[s][r, :]
            plsc.store_scatter(
                tbuf[s], [iota, jnp.full((16,), r, jnp.int32)], prod)
        q0 = p * 512 + base8
        pltpu.async_copy(
            tbuf[s].at[pl.ds(0, 8), pl.ds(0, 128)],
            out_hbm.at[pl.ds(q0, 8), :], semo[s])
        pltpu.async_copy(
            tbuf[s].at[pl.ds(8, 8), pl.ds(0, 128)],
            out_hbm.at[pl.ds(q0 + 256, 8), :], semo[s])

    def wait_out(p, s):
        q0 = p * 512 + base8
        pltpu.make_async_copy(
            tbuf[s].at[pl.ds(0, 8), pl.ds(0, 128)],
            out_hbm.at[pl.ds(q0, 8), :], semo[s]).wait()
        pltpu.make_async_copy(
            tbuf[s].at[pl.ds(8, 8), pl.ds(0, 128)],
            out_hbm.at[pl.ds(q0 + 256, 8), :], semo[s]).wait()

    for s in range(_NBUF):
        issue_static(s, s)

    for s in range(_NBUF):
        wait_gathers(s)
        multiply_and_flush(s, s)
        issue_static(s + _NBUF, s)

    def group_body(g, carry):
        for s in range(_NBUF):
            p = g * _NBUF + s
            wait_gathers(s)
            wait_out(p, s)
            multiply_and_flush(p, s)
            issue_dyn(p + _NBUF, s)
        return carry

    lax.fori_loop(1, _NGROUPS, group_body, 0)
    for s in range(_NBUF):
        wait_gathers(s)
        wait_out(s, s)


def kernel(x, W):
    b, f = x.shape
    assert f == _F
    wf = _reformat_w(W)
    x32 = x.astype(jnp.int32)
    info = plsc.get_sparse_core_info()
    nw = info.num_cores * info.num_subcores
    c = b // nw
    nrows = b * _NPAIRS * _D // 128
    mesh = plsc.VectorSubcoreMesh(core_axis_name="c", subcore_axis_name="s")
    scratch = [
        pltpu.VMEM((c, _F), jnp.int32),
        pltpu.VMEM((_PAIR_PAD,), jnp.int32),
        pltpu.VMEM((_PAIR_PAD,), jnp.int32),
    ]
    scratch += [pltpu.VMEM((c,), jnp.int32) for _ in range(2 * _NBUF)]
    scratch += [pltpu.VMEM((c, _D), jnp.float32) for _ in range(2 * _NBUF)]
    scratch += [pltpu.VMEM((_D, 129), jnp.float32) for _ in range(_NBUF)]
    scratch += [pltpu.SemaphoreType.DMA for _ in range(3 * _NBUF)]
    kfn = functools.partial(
        pl.kernel,
        out_type=jax.ShapeDtypeStruct((nrows, 128), jnp.float32),
        mesh=mesh,
        compiler_params=pltpu.CompilerParams(needs_layout_passes=False,
                                             use_tc_tiling_on_sc=False),
        scratch_types=scratch,
    )(_sc_body)
    out2d = kfn(x32, wf, jnp.asarray(_pi), jnp.asarray(_pj))
    return (out2d.reshape(_NPAIRS, 2, b // 128, 8, 128)
            .transpose(2, 4, 0, 1, 3)
            .reshape(b, _NPAIRS, _D))
